# baseline (speedup 1.0000x reference)
import jax
import jax.numpy as jnp
from jax import lax
from jax.experimental import pallas as pl
from jax.experimental.pallas import tpu as pltpu
from jax.experimental.pallas import tpu_sc as plsc

N = 10000
NPAD = 10112
E = 320000
D = 128
H = 64
HH = 32
B = 256
NC, NS = 2, 16
CH = 256
EROWS = 1280
RPT = EROWS // NS
G = 8
G2 = 5
NZ = NPAD // NS
NT = N // NS
RB = 1264

f32 = jnp.float32
i32 = jnp.int32

_MESH = plsc.VectorSubcoreMesh(
    core_axis_name="c", subcore_axis_name="s", num_cores=NC, num_subcores=NS)
_SC_PARAMS = pltpu.CompilerParams(use_tc_tiling_on_sc=False)


def _cnt_body(d_ss, d_gg, d_gs, d_sg, zeros16,
              deg_ss, deg_gg, cnt_gs, cnt_sg,
              idx0, idx1, ones, acc, is0, is1, sa0, sa1):
    c = lax.axis_index("c")
    t = lax.axis_index("s")
    zsl = pl.ds(t * NZ, NZ)
    ngh = RPT // (2 * G)

    def fill(i, carry):
        ones[i, :] = jnp.ones((16,), f32)
        return carry
    lax.fori_loop(0, CH, fill, 0)

    def run(dst2d, out):
        pltpu.sync_copy(zeros16.at[zsl], acc.at[zsl])
        plsc.subcore_barrier()
        base = t * RPT

        def drain_i(sem):
            pltpu.make_async_copy(dst2d.at[pl.ds(0, G)], idx0, sem).wait()

        def drain_s(sem):
            d = pltpu.make_async_copy(zeros16.at[pl.ds(0, CH)], ones, sem)
            for _ in range(G):
                d.wait()

        def scatters(ibuf, sem):
            for j in range(G):
                pltpu.async_copy(ones, acc.at[ibuf.at[j]], sem, add=True)

        pltpu.async_copy(dst2d.at[pl.ds(base, G)], idx0, is0)

        def body(i, carry):
            @pl.when(i > 0)
            def _():
                drain_s(sa1)
            pltpu.async_copy(
                dst2d.at[pl.ds(base + (2 * i + 1) * G, G)], idx1, is1)
            drain_i(is0)
            scatters(idx0, sa0)

            @pl.when(i < ngh - 1)
            def _():
                drain_s(sa0)
                pltpu.async_copy(
                    dst2d.at[pl.ds(base + (2 * i + 2) * G, G)], idx0, is0)
            drain_i(is1)
            scatters(idx1, sa1)
            return carry
        lax.fori_loop(0, ngh, body, 0)
        drain_s(sa0)
        drain_s(sa1)
        plsc.subcore_barrier()
        pltpu.sync_copy(acc.at[zsl], out.at[zsl])

    @pl.when(c == 0)
    def _():
        run(d_ss, deg_ss)
        run(d_gs, cnt_gs)

    @pl.when(c == 1)
    def _():
        run(d_gg, deg_gg)
        run(d_sg, cnt_sg)


_cnt_call = pl.kernel(
    _cnt_body,
    out_type=[jax.ShapeDtypeStruct((NPAD, 16), f32)] * 4,
    mesh=_MESH,
    compiler_params=_SC_PARAMS,
    scratch_types=[
        pltpu.VMEM((G, CH), i32),
        pltpu.VMEM((G, CH), i32),
        pltpu.VMEM((CH, 16), f32),
        pltpu.VMEM_SHARED((NPAD, 16), f32),
        pltpu.SemaphoreType.DMA,
        pltpu.SemaphoreType.DMA,
        pltpu.SemaphoreType.DMA,
        pltpu.SemaphoreType.DMA,
    ],
)


def _agg_body(y_ss_lo, y_ss_hi, s_ss, d_ss, y_gs_lo, y_gs_hi, s_gs, d_gs,
              y_gg_lo, y_gg_hi, s_gg, d_gg, y_sg_lo, y_sg_hi, s_sg, d_sg,
              zeros32,
              a_ss_lo, a_ss_hi, a_gs_lo, a_gs_hi,
              a_gg_lo, a_gg_hi, a_sg_lo, a_sg_hi,
              src0, dst0, src1, dst1, rows0, rows1, acc, tab,
              gs0, gs1, ss0, ss1):
    c = lax.axis_index("c")
    t = lax.axis_index("s")
    zsl = pl.ds(t * NZ, NZ)
    tsl = pl.ds(t * NT, NT)
    ngh = RPT // (2 * G2)

    def run(table, src2d, dst2d, out):
        pltpu.sync_copy(zeros32.at[zsl], acc.at[zsl])
        pltpu.sync_copy(table.at[tsl], tab.at[tsl])
        plsc.subcore_barrier()
        base = t * RPT

        def drain(sem):
            d = pltpu.make_async_copy(table.at[pl.ds(0, CH)], rows0.at[0], sem)
            for _ in range(G2):
                d.wait()

        def load_idx(goff, sbuf, dbuf):
            pltpu.sync_copy(src2d.at[pl.ds(base + goff, G2)], sbuf)
            pltpu.sync_copy(dst2d.at[pl.ds(base + goff, G2)], dbuf)

        def gathers(sbuf, rbuf, sem):
            for j in range(G2):
                pltpu.async_copy(tab.at[sbuf.at[j]], rbuf.at[j], sem)

        def scatters(dbuf, rbuf, sem):
            for j in range(G2):
                pltpu.async_copy(rbuf.at[j], acc.at[dbuf.at[j]], sem, add=True)

        load_idx(0, src0, dst0)
        gathers(src0, rows0, gs0)

        def body(i, carry):
            @pl.when(i > 0)
            def _():
                drain(ss1)
            load_idx((2 * i + 1) * G2, src1, dst1)
            gathers(src1, rows1, gs1)
            drain(gs0)
            scatters(dst0, rows0, ss0)

            @pl.when(i < ngh - 1)
            def _():
                drain(ss0)
                load_idx((2 * i + 2) * G2, src0, dst0)
                gathers(src0, rows0, gs0)
            drain(gs1)
            scatters(dst1, rows1, ss1)
            return carry
        lax.fori_loop(0, ngh, body, 0)
        drain(ss0)
        drain(ss1)
        plsc.subcore_barrier()
        pltpu.sync_copy(acc.at[zsl], out.at[zsl])

    @pl.when(c == 0)
    def _():
        run(y_ss_lo, s_ss, d_ss, a_ss_lo)
        run(y_ss_hi, s_ss, d_ss, a_ss_hi)
        run(y_gs_lo, s_gs, d_gs, a_gs_lo)
        run(y_gs_hi, s_gs, d_gs, a_gs_hi)

    @pl.when(c == 1)
    def _():
        run(y_gg_lo, s_gg, d_gg, a_gg_lo)
        run(y_gg_hi, s_gg, d_gg, a_gg_hi)
        run(y_sg_lo, s_sg, d_sg, a_sg_lo)
        run(y_sg_hi, s_sg, d_sg, a_sg_hi)


_agg_call = pl.kernel(
    _agg_body,
    out_type=[jax.ShapeDtypeStruct((NPAD, HH), f32)] * 8,
    mesh=_MESH,
    compiler_params=_SC_PARAMS,
    scratch_types=[
        pltpu.VMEM((G2, CH), i32),
        pltpu.VMEM((G2, CH), i32),
        pltpu.VMEM((G2, CH), i32),
        pltpu.VMEM((G2, CH), i32),
        pltpu.VMEM((G2, CH, HH), f32),
        pltpu.VMEM((G2, CH, HH), f32),
        pltpu.VMEM_SHARED((NPAD, HH), f32),
        pltpu.VMEM_SHARED((N, HH), f32),
        pltpu.SemaphoreType.DMA,
        pltpu.SemaphoreType.DMA,
        pltpu.SemaphoreType.DMA,
        pltpu.SemaphoreType.DMA,
    ],
)


def _mm0_body(xs, xg, ws, wg, xw_s, xw_g):
    xw_s[...] = jnp.dot(xs[...], ws[...], preferred_element_type=f32)
    xw_g[...] = jnp.dot(xg[...], wg[...], preferred_element_type=f32)


def _prep0_body(xws_ref, xwg_ref, degss, deggg,
                y_ss_lo, y_ss_hi, r_s, y_sg_lo, y_sg_hi,
                y_gg_lo, y_gg_hi, r_g, y_gs_lo, y_gs_hi):
    dinv_s = lax.rsqrt(degss[...][:, 0:1] + 1.0)
    dinv_g = lax.rsqrt(deggg[...][:, 0:1] + 1.0)
    xws = xws_ref[...]
    xwg = xwg_ref[...]
    y_ss_lo[...] = dinv_s * xws[:, 0:HH]
    y_ss_hi[...] = dinv_s * xws[:, HH:H]
    r_s[...] = xws[:, H:2 * H]
    y_sg_lo[...] = xws[:, 2 * H:2 * H + HH]
    y_sg_hi[...] = xws[:, 2 * H + HH:3 * H]
    y_gg_lo[...] = dinv_g * xwg[:, 0:HH]
    y_gg_hi[...] = dinv_g * xwg[:, HH:H]
    r_g[...] = xwg[:, H:2 * H]
    y_gs_lo[...] = xwg[:, 2 * H:2 * H + HH]
    y_gs_hi[...] = xwg[:, 2 * H + HH:3 * H]


def _comb(a_ss, a_gs, a_gg, a_sg, yss, rs, ygg, rg,
          degss, deggg, cntgs, cntsg, bs1, bs2, bg1, bg2):
    dinv_s = lax.rsqrt(degss[...][:, 0:1] + 1.0)
    dinv_g = lax.rsqrt(deggg[...][:, 0:1] + 1.0)
    ic_gs = 1.0 / jnp.maximum(cntgs[...][:, 0:1], 1.0)
    ic_sg = 1.0 / jnp.maximum(cntsg[...][:, 0:1], 1.0)
    ns = 0.5 * (dinv_s * (a_ss + yss) + ic_gs * a_gs
                + rs[...] + bs1[...] + bs2[...])
    ng = 0.5 * (dinv_g * (a_gg + ygg) + ic_sg * a_sg
                + rg[...] + bg1[...] + bg2[...])
    return ns, ng, dinv_s, dinv_g


def _cat2(lo, hi):
    return jnp.concatenate([lo[...], hi[...]], axis=1)


def _comb_prep_body(al_ss, ah_ss, al_gs, ah_gs, al_gg, ah_gg, al_sg, ah_sg,
                    yl_ss, yh_ss, rs, yl_gg, yh_gg, rg,
                    degss, deggg, cntgs, cntsg, bs1, bs2, bg1, bg2, ws, wg,
                    y_ss_lo, y_ss_hi, r_s1, y_sg_lo, y_sg_hi,
                    y_gg_lo, y_gg_hi, r_g1, y_gs_lo, y_gs_hi):
    ns, ng, dinv_s, dinv_g = _comb(
        _cat2(al_ss, ah_ss), _cat2(al_gs, ah_gs),
        _cat2(al_gg, ah_gg), _cat2(al_sg, ah_sg),
        _cat2(yl_ss, yh_ss), rs, _cat2(yl_gg, yh_gg), rg,
        degss, deggg, cntgs, cntsg, bs1, bs2, bg1, bg2)
    xws = jnp.dot(ns, ws[...], preferred_element_type=f32)
    xwg = jnp.dot(ng, wg[...], preferred_element_type=f32)
    y_ss_lo[...] = dinv_s * xws[:, 0:HH]
    y_ss_hi[...] = dinv_s * xws[:, HH:H]
    r_s1[...] = xws[:, H:2 * H]
    y_sg_lo[...] = xws[:, 2 * H:2 * H + HH]
    y_sg_hi[...] = xws[:, 2 * H + HH:3 * H]
    y_gg_lo[...] = dinv_g * xwg[:, 0:HH]
    y_gg_hi[...] = dinv_g * xwg[:, HH:H]
    r_g1[...] = xwg[:, H:2 * H]
    y_gs_lo[...] = xwg[:, 2 * H:2 * H + HH]
    y_gs_hi[...] = xwg[:, 2 * H + HH:3 * H]


def _comb_pool_body(al_ss, ah_ss, al_gs, ah_gs, al_gg, ah_gg, al_sg, ah_sg,
                    yl_ss, yh_ss, rs, yl_gg, yh_gg, rg,
                    degss, deggg, cntgs, cntsg, bs1, bs2, bg1, bg2,
                    bidx_s, bidx_g, dep, wsg, wdb,
                    out, ps, cs, pg, cg):
    i = pl.program_id(0)
    ns, ng, _, _ = _comb(
        _cat2(al_ss, ah_ss), _cat2(al_gs, ah_gs),
        _cat2(al_gg, ah_gg), _cat2(al_sg, ah_sg),
        _cat2(yl_ss, yh_ss), rs, _cat2(yl_gg, yh_gg), rg,
        degss, deggg, cntgs, cntsg, bs1, bs2, bg1, bg2)
    segs = lax.broadcasted_iota(i32, (1, B), 1)
    oh_s = (bidx_s[...] == segs).astype(f32)
    oh_g = (bidx_g[...] == segs).astype(f32)
    dn = (((0,), (0,)), ((), ()))
    ones_col = jnp.ones((RB, 1), f32)
    psum = lax.dot_general(oh_s, ns, dn, preferred_element_type=f32)
    pcnt = lax.dot_general(oh_s, ones_col, dn, preferred_element_type=f32)
    gsum = lax.dot_general(oh_g, ng, dn, preferred_element_type=f32)
    gcnt = lax.dot_general(oh_g, ones_col, dn, preferred_element_type=f32)

    @pl.when(i == 0)
    def _():
        ps[...] = psum
        cs[...] = pcnt
        pg[...] = gsum
        cg[...] = gcnt

    @pl.when(i > 0)
    def _():
        ps[...] += psum
        cs[...] += pcnt
        pg[...] += gsum
        cg[...] += gcnt

    @pl.when(i == NPAD // RB - 1)
    def _():
        s = ps[...] / jnp.maximum(cs[...], 1.0)
        g = pg[...] / jnp.maximum(cg[...], 1.0)
        w_s = wsg[...][0:1, :]
        w_g = wsg[...][1:2, :]
        acc = (jnp.sum(s * w_s, axis=1, keepdims=True)
               + jnp.sum(g * w_g, axis=1, keepdims=True))
        out[...] = acc + dep[...] * wdb[0] + wdb[1]


def _row_spec(width):
    return pl.BlockSpec((RB, width), lambda i: (i, 0))


def _full_spec(shape):
    return pl.BlockSpec(shape, lambda i: (0, 0))


_GRID = (NPAD // RB,)

_PREP_OUT_SPECS = [_row_spec(HH), _row_spec(HH), _row_spec(H),
                   _row_spec(HH), _row_spec(HH)] * 2
_PREP_OUT_SHAPE = [jax.ShapeDtypeStruct((NPAD, HH), f32),
                   jax.ShapeDtypeStruct((NPAD, HH), f32),
                   jax.ShapeDtypeStruct((NPAD, H), f32),
                   jax.ShapeDtypeStruct((NPAD, HH), f32),
                   jax.ShapeDtypeStruct((NPAD, HH), f32)] * 2


def _mm0_call(xs, xg, ws, wg):
    return pl.pallas_call(
        _mm0_body,
        grid=_GRID,
        in_specs=[_row_spec(D), _row_spec(D),
                  _full_spec((D, 3 * H)), _full_spec((D, 3 * H))],
        out_specs=[_row_spec(3 * H)] * 2,
        out_shape=[jax.ShapeDtypeStruct((NPAD, 3 * H), f32)] * 2,
    )(xs, xg, ws, wg)


def _prep0_call(xws, xwg, degss, deggg):
    return pl.pallas_call(
        _prep0_body,
        grid=_GRID,
        in_specs=[_row_spec(3 * H), _row_spec(3 * H),
                  _row_spec(16), _row_spec(16)],
        out_specs=_PREP_OUT_SPECS,
        out_shape=_PREP_OUT_SHAPE,
    )(xws, xwg, degss, deggg)


_COMB_IN_SPECS = ([_row_spec(HH)] * 8
                  + [_row_spec(HH), _row_spec(HH), _row_spec(H)] * 2
                  + [_row_spec(16)] * 4 + [_full_spec((1, H))] * 4)


def _comb_prep_call(*args):
    return pl.pallas_call(
        _comb_prep_body,
        grid=_GRID,
        in_specs=_COMB_IN_SPECS + [_full_spec((H, 3 * H))] * 2,
        out_specs=_PREP_OUT_SPECS,
        out_shape=_PREP_OUT_SHAPE,
    )(*args)


def _comb_pool_call(*args):
    return pl.pallas_call(
        _comb_pool_body,
        grid=_GRID,
        in_specs=_COMB_IN_SPECS + [
            _row_spec(1),
            _row_spec(1),
            _full_spec((B, 1)),
            _full_spec((2, H)),
            pl.BlockSpec(memory_space=pltpu.SMEM),
        ],
        out_specs=pl.BlockSpec((B, 1), lambda i: (0, 0)),
        out_shape=jax.ShapeDtypeStruct((B, 1), f32),
        scratch_shapes=[
            pltpu.VMEM((B, H), f32),
            pltpu.VMEM((B, 1), f32),
            pltpu.VMEM((B, H), f32),
            pltpu.VMEM((B, 1), f32),
        ],
    )(*args)


def _pad_idx(v, fill, rows):
    v = v.astype(i32)
    pad = rows * CH - v.shape[0]
    v = jnp.concatenate([v, jnp.full((pad,), fill, i32)])
    return v.reshape(rows, CH)


def _pad_rows(x):
    return jnp.concatenate(
        [x.astype(f32), jnp.zeros((NPAD - N, x.shape[1]), f32)])


def kernel(x_state, x_goal, ei_ss, ei_gg, ei_sg, ei_gs, batch_state,
           batch_goal, depth,
           W_gcn_s_0, b_gcn_s_0, W_gcn_g_0, b_gcn_g_0, Wl_sg_0, bl_sg_0,
           Wr_sg_0, Wl_gs_0, bl_gs_0, Wr_gs_0,
           W_gcn_s_1, b_gcn_s_1, W_gcn_g_1, b_gcn_g_1, Wl_sg_1, bl_sg_1,
           Wr_sg_1, Wl_gs_1, bl_gs_1, Wr_gs_1, W_out, b_out):
    s_ss = _pad_idx(ei_ss[0], 0, EROWS)
    d_ss = _pad_idx(ei_ss[1], N, EROWS)
    s_gg = _pad_idx(ei_gg[0], 0, EROWS)
    d_gg = _pad_idx(ei_gg[1], N, EROWS)
    s_sg = _pad_idx(ei_sg[0], 0, EROWS)
    d_sg = _pad_idx(ei_sg[1], N, EROWS)
    s_gs = _pad_idx(ei_gs[0], 0, EROWS)
    d_gs = _pad_idx(ei_gs[1], N, EROWS)

    bidx_s = jnp.concatenate(
        [batch_state.astype(i32), jnp.full((NPAD - N,), B, i32)]).reshape(
            NPAD, 1)
    bidx_g = jnp.concatenate(
        [batch_goal.astype(i32), jnp.full((NPAD - N,), B, i32)]).reshape(
            NPAD, 1)

    zeros32 = jnp.zeros((NPAD, HH), f32)
    zeros16 = jnp.zeros((NPAD, 16), f32)

    ws0 = jnp.concatenate([W_gcn_s_0, Wr_gs_0, Wl_sg_0], axis=1)
    wg0 = jnp.concatenate([W_gcn_g_0, Wr_sg_0, Wl_gs_0], axis=1)
    ws1 = jnp.concatenate([W_gcn_s_1, Wr_gs_1, Wl_sg_1], axis=1)
    wg1 = jnp.concatenate([W_gcn_g_1, Wr_sg_1, Wl_gs_1], axis=1)
    bs1_0 = b_gcn_s_0.reshape(1, H)
    bs2_0 = bl_gs_0.reshape(1, H)
    bg1_0 = b_gcn_g_0.reshape(1, H)
    bg2_0 = bl_sg_0.reshape(1, H)
    bs1_1 = b_gcn_s_1.reshape(1, H)
    bs2_1 = bl_gs_1.reshape(1, H)
    bg1_1 = b_gcn_g_1.reshape(1, H)
    bg2_1 = bl_sg_1.reshape(1, H)

    xw_s0, xw_g0 = _mm0_call(
        _pad_rows(x_state), _pad_rows(x_goal), ws0, wg0)

    deg_ss, deg_gg, cnt_gs, cnt_sg = _cnt_call(d_ss, d_gg, d_gs, d_sg, zeros16)

    (yl_ss0, yh_ss0, r_s0, yl_sg0, yh_sg0,
     yl_gg0, yh_gg0, r_g0, yl_gs0, yh_gs0) = _prep0_call(
        xw_s0, xw_g0, deg_ss, deg_gg)

    (al_ss0, ah_ss0, al_gs0, ah_gs0,
     al_gg0, ah_gg0, al_sg0, ah_sg0) = _agg_call(
        yl_ss0, yh_ss0, s_ss, d_ss, yl_gs0, yh_gs0, s_gs, d_gs,
        yl_gg0, yh_gg0, s_gg, d_gg, yl_sg0, yh_sg0, s_sg, d_sg, zeros32)

    (yl_ss1, yh_ss1, r_s1, yl_sg1, yh_sg1,
     yl_gg1, yh_gg1, r_g1, yl_gs1, yh_gs1) = _comb_prep_call(
        al_ss0, ah_ss0, al_gs0, ah_gs0, al_gg0, ah_gg0, al_sg0, ah_sg0,
        yl_ss0, yh_ss0, r_s0, yl_gg0, yh_gg0, r_g0,
        deg_ss, deg_gg, cnt_gs, cnt_sg, bs1_0, bs2_0, bg1_0, bg2_0, ws1, wg1)

    (al_ss1, ah_ss1, al_gs1, ah_gs1,
     al_gg1, ah_gg1, al_sg1, ah_sg1) = _agg_call(
        yl_ss1, yh_ss1, s_ss, d_ss, yl_gs1, yh_gs1, s_gs, d_gs,
        yl_gg1, yh_gg1, s_gg, d_gg, yl_sg1, yh_sg1, s_sg, d_sg, zeros32)

    wsg = jnp.stack([W_out[0:H, 0], W_out[H:2 * H, 0]])
    wdb = jnp.stack([W_out[2 * H, 0], b_out[0]])
    out = _comb_pool_call(
        al_ss1, ah_ss1, al_gs1, ah_gs1, al_gg1, ah_gg1, al_sg1, ah_sg1,
        yl_ss1, yh_ss1, r_s1, yl_gg1, yh_gg1, r_g1,
        deg_ss, deg_gg, cnt_gs, cnt_sg, bs1_1, bs2_1, bg1_1, bg2_1,
        bidx_s, bidx_g, depth.astype(f32).reshape(B, 1), wsg, wdb)
    return out[:, 0]


# speedup vs baseline: 1.0044x; 1.0044x over previous
import jax
import jax.numpy as jnp
from jax import lax
from jax.experimental import pallas as pl
from jax.experimental.pallas import tpu as pltpu
from jax.experimental.pallas import tpu_sc as plsc

N = 10000
NPAD = 10112
E = 320000
D = 128
H = 64
HH = 32
B = 256
NC, NS = 2, 16
CH = 256
EROWS = 1280
RPT = EROWS // NS
G = 8
G2 = 5
NZ = NPAD // NS
NT = N // NS
RB = 1264

f32 = jnp.float32
i32 = jnp.int32

_MESH = plsc.VectorSubcoreMesh(
    core_axis_name="c", subcore_axis_name="s", num_cores=NC, num_subcores=NS)
_SC_PARAMS = pltpu.CompilerParams(use_tc_tiling_on_sc=False)


def _cnt_body(d_ss, d_gg, d_gs, d_sg, zeros16,
              deg_ss, deg_gg, cnt_gs, cnt_sg,
              idx0, idx1, ones, acc, is0, is1, sa0, sa1):
    c = lax.axis_index("c")
    t = lax.axis_index("s")
    zsl = pl.ds(t * NZ, NZ)
    ngh = RPT // (2 * G)

    def fill(i, carry):
        ones[i, :] = jnp.ones((16,), f32)
        return carry
    lax.fori_loop(0, CH, fill, 0)

    def run(dst2d, out):
        pltpu.sync_copy(zeros16.at[zsl], acc.at[zsl])
        plsc.subcore_barrier()
        base = t * RPT

        def drain_i(sem):
            pltpu.make_async_copy(dst2d.at[pl.ds(0, G)], idx0, sem).wait()

        def drain_s(sem):
            d = pltpu.make_async_copy(zeros16.at[pl.ds(0, CH)], ones, sem)
            for _ in range(G):
                d.wait()

        def scatters(ibuf, sem):
            for j in range(G):
                pltpu.async_copy(ones, acc.at[ibuf.at[j]], sem, add=True)

        pltpu.async_copy(dst2d.at[pl.ds(base, G)], idx0, is0)

        def body(i, carry):
            @pl.when(i > 0)
            def _():
                drain_s(sa1)
            pltpu.async_copy(
                dst2d.at[pl.ds(base + (2 * i + 1) * G, G)], idx1, is1)
            drain_i(is0)
            scatters(idx0, sa0)

            @pl.when(i < ngh - 1)
            def _():
                drain_s(sa0)
                pltpu.async_copy(
                    dst2d.at[pl.ds(base + (2 * i + 2) * G, G)], idx0, is0)
            drain_i(is1)
            scatters(idx1, sa1)
            return carry
        lax.fori_loop(0, ngh, body, 0)
        drain_s(sa0)
        drain_s(sa1)
        plsc.subcore_barrier()
        pltpu.sync_copy(acc.at[zsl], out.at[zsl])

    @pl.when(c == 0)
    def _():
        run(d_ss, deg_ss)
        run(d_gs, cnt_gs)

    @pl.when(c == 1)
    def _():
        run(d_gg, deg_gg)
        run(d_sg, cnt_sg)


_cnt_call = pl.kernel(
    _cnt_body,
    out_type=[jax.ShapeDtypeStruct((NPAD, 16), f32)] * 4,
    mesh=_MESH,
    compiler_params=_SC_PARAMS,
    scratch_types=[
        pltpu.VMEM((G, CH), i32),
        pltpu.VMEM((G, CH), i32),
        pltpu.VMEM((CH, 16), f32),
        pltpu.VMEM_SHARED((NPAD, 16), f32),
        pltpu.SemaphoreType.DMA,
        pltpu.SemaphoreType.DMA,
        pltpu.SemaphoreType.DMA,
        pltpu.SemaphoreType.DMA,
    ],
)


def _agg_body(y_ss_lo, y_ss_hi, s_ss, d_ss, y_gs_lo, y_gs_hi, s_gs, d_gs,
              y_gg_lo, y_gg_hi, s_gg, d_gg, y_sg_lo, y_sg_hi, s_sg, d_sg,
              zeros32,
              a_ss_lo, a_ss_hi, a_gs_lo, a_gs_hi,
              a_gg_lo, a_gg_hi, a_sg_lo, a_sg_hi,
              src0, dst0, src1, dst1, rows0, rows1, acc, tab,
              gs0, gs1, ss0, ss1):
    c = lax.axis_index("c")
    t = lax.axis_index("s")
    zsl = pl.ds(t * NZ, NZ)
    tsl = pl.ds(t * NT, NT)
    ngh = RPT // (2 * G2)

    def run(table, src2d, dst2d, out):
        pltpu.sync_copy(zeros32.at[zsl], acc.at[zsl])
        pltpu.sync_copy(table.at[tsl], tab.at[tsl])
        plsc.subcore_barrier()
        base = t * RPT

        def drain(sem):
            d = pltpu.make_async_copy(table.at[pl.ds(0, CH)], rows0.at[0], sem)
            for _ in range(G2):
                d.wait()

        def load_idx(goff, sbuf, dbuf):
            pltpu.sync_copy(src2d.at[pl.ds(base + goff, G2)], sbuf)
            pltpu.sync_copy(dst2d.at[pl.ds(base + goff, G2)], dbuf)

        def gathers(sbuf, rbuf, sem):
            for j in range(G2):
                pltpu.async_copy(tab.at[sbuf.at[j]], rbuf.at[j], sem)

        def scatters(dbuf, rbuf, sem):
            for j in range(G2):
                pltpu.async_copy(rbuf.at[j], acc.at[dbuf.at[j]], sem, add=True)

        load_idx(0, src0, dst0)
        gathers(src0, rows0, gs0)

        def body(i, carry):
            @pl.when(i > 0)
            def _():
                drain(ss1)
            load_idx((2 * i + 1) * G2, src1, dst1)
            gathers(src1, rows1, gs1)
            drain(gs0)
            scatters(dst0, rows0, ss0)

            @pl.when(i < ngh - 1)
            def _():
                drain(ss0)
                load_idx((2 * i + 2) * G2, src0, dst0)
                gathers(src0, rows0, gs0)
            drain(gs1)
            scatters(dst1, rows1, ss1)
            return carry
        lax.fori_loop(0, ngh, body, 0)
        drain(ss0)
        drain(ss1)
        plsc.subcore_barrier()
        pltpu.sync_copy(acc.at[zsl], out.at[zsl])

    @pl.when(c == 0)
    def _():
        run(y_ss_lo, s_ss, d_ss, a_ss_lo)
        run(y_ss_hi, s_ss, d_ss, a_ss_hi)
        run(y_gs_lo, s_gs, d_gs, a_gs_lo)
        run(y_gs_hi, s_gs, d_gs, a_gs_hi)

    @pl.when(c == 1)
    def _():
        run(y_gg_lo, s_gg, d_gg, a_gg_lo)
        run(y_gg_hi, s_gg, d_gg, a_gg_hi)
        run(y_sg_lo, s_sg, d_sg, a_sg_lo)
        run(y_sg_hi, s_sg, d_sg, a_sg_hi)


_agg_call = pl.kernel(
    _agg_body,
    out_type=[jax.ShapeDtypeStruct((NPAD, HH), f32)] * 8,
    mesh=_MESH,
    compiler_params=_SC_PARAMS,
    scratch_types=[
        pltpu.VMEM((G2, CH), i32),
        pltpu.VMEM((G2, CH), i32),
        pltpu.VMEM((G2, CH), i32),
        pltpu.VMEM((G2, CH), i32),
        pltpu.VMEM((G2, CH, HH), f32),
        pltpu.VMEM((G2, CH, HH), f32),
        pltpu.VMEM_SHARED((NPAD, HH), f32),
        pltpu.VMEM_SHARED((N, HH), f32),
        pltpu.SemaphoreType.DMA,
        pltpu.SemaphoreType.DMA,
        pltpu.SemaphoreType.DMA,
        pltpu.SemaphoreType.DMA,
    ],
)


def _prep0_body(xs, xg, degss, deggg, ws, wg,
                y_ss_lo, y_ss_hi, r_s, y_sg_lo, y_sg_hi,
                y_gg_lo, y_gg_hi, r_g, y_gs_lo, y_gs_hi):
    dinv_s = lax.rsqrt(degss[...][:, 0:1] + 1.0)
    dinv_g = lax.rsqrt(deggg[...][:, 0:1] + 1.0)
    xws = jnp.dot(xs[...], ws[...], preferred_element_type=f32)
    xwg = jnp.dot(xg[...], wg[...], preferred_element_type=f32)
    y_ss_lo[...] = dinv_s * xws[:, 0:HH]
    y_ss_hi[...] = dinv_s * xws[:, HH:H]
    r_s[...] = xws[:, H:2 * H]
    y_sg_lo[...] = xws[:, 2 * H:2 * H + HH]
    y_sg_hi[...] = xws[:, 2 * H + HH:3 * H]
    y_gg_lo[...] = dinv_g * xwg[:, 0:HH]
    y_gg_hi[...] = dinv_g * xwg[:, HH:H]
    r_g[...] = xwg[:, H:2 * H]
    y_gs_lo[...] = xwg[:, 2 * H:2 * H + HH]
    y_gs_hi[...] = xwg[:, 2 * H + HH:3 * H]


def _comb(a_ss, a_gs, a_gg, a_sg, yss, rs, ygg, rg,
          degss, deggg, cntgs, cntsg, bs1, bs2, bg1, bg2):
    dinv_s = lax.rsqrt(degss[...][:, 0:1] + 1.0)
    dinv_g = lax.rsqrt(deggg[...][:, 0:1] + 1.0)
    ic_gs = 1.0 / jnp.maximum(cntgs[...][:, 0:1], 1.0)
    ic_sg = 1.0 / jnp.maximum(cntsg[...][:, 0:1], 1.0)
    ns = 0.5 * (dinv_s * (a_ss + yss) + ic_gs * a_gs
                + rs[...] + bs1[...] + bs2[...])
    ng = 0.5 * (dinv_g * (a_gg + ygg) + ic_sg * a_sg
                + rg[...] + bg1[...] + bg2[...])
    return ns, ng, dinv_s, dinv_g


def _cat2(lo, hi):
    return jnp.concatenate([lo[...], hi[...]], axis=1)


def _comb_prep_body(al_ss, ah_ss, al_gs, ah_gs, al_gg, ah_gg, al_sg, ah_sg,
                    yl_ss, yh_ss, rs, yl_gg, yh_gg, rg,
                    degss, deggg, cntgs, cntsg, bs1, bs2, bg1, bg2, ws, wg,
                    y_ss_lo, y_ss_hi, r_s1, y_sg_lo, y_sg_hi,
                    y_gg_lo, y_gg_hi, r_g1, y_gs_lo, y_gs_hi):
    ns, ng, dinv_s, dinv_g = _comb(
        _cat2(al_ss, ah_ss), _cat2(al_gs, ah_gs),
        _cat2(al_gg, ah_gg), _cat2(al_sg, ah_sg),
        _cat2(yl_ss, yh_ss), rs, _cat2(yl_gg, yh_gg), rg,
        degss, deggg, cntgs, cntsg, bs1, bs2, bg1, bg2)
    xws = jnp.dot(ns, ws[...], preferred_element_type=f32)
    xwg = jnp.dot(ng, wg[...], preferred_element_type=f32)
    y_ss_lo[...] = dinv_s * xws[:, 0:HH]
    y_ss_hi[...] = dinv_s * xws[:, HH:H]
    r_s1[...] = xws[:, H:2 * H]
    y_sg_lo[...] = xws[:, 2 * H:2 * H + HH]
    y_sg_hi[...] = xws[:, 2 * H + HH:3 * H]
    y_gg_lo[...] = dinv_g * xwg[:, 0:HH]
    y_gg_hi[...] = dinv_g * xwg[:, HH:H]
    r_g1[...] = xwg[:, H:2 * H]
    y_gs_lo[...] = xwg[:, 2 * H:2 * H + HH]
    y_gs_hi[...] = xwg[:, 2 * H + HH:3 * H]


def _comb_pool_body(al_ss, ah_ss, al_gs, ah_gs, al_gg, ah_gg, al_sg, ah_sg,
                    yl_ss, yh_ss, rs, yl_gg, yh_gg, rg,
                    degss, deggg, cntgs, cntsg, bs1, bs2, bg1, bg2,
                    bidx_s, bidx_g, dep, wsg, wdb,
                    out, ps, cs, pg, cg):
    i = pl.program_id(0)
    ns, ng, _, _ = _comb(
        _cat2(al_ss, ah_ss), _cat2(al_gs, ah_gs),
        _cat2(al_gg, ah_gg), _cat2(al_sg, ah_sg),
        _cat2(yl_ss, yh_ss), rs, _cat2(yl_gg, yh_gg), rg,
        degss, deggg, cntgs, cntsg, bs1, bs2, bg1, bg2)
    segs = lax.broadcasted_iota(i32, (1, B), 1)
    oh_s = (bidx_s[...] == segs).astype(f32)
    oh_g = (bidx_g[...] == segs).astype(f32)
    dn = (((0,), (0,)), ((), ()))
    ones_col = jnp.ones((RB, 1), f32)
    psum = lax.dot_general(oh_s, ns, dn, preferred_element_type=f32)
    pcnt = lax.dot_general(oh_s, ones_col, dn, preferred_element_type=f32)
    gsum = lax.dot_general(oh_g, ng, dn, preferred_element_type=f32)
    gcnt = lax.dot_general(oh_g, ones_col, dn, preferred_element_type=f32)

    @pl.when(i == 0)
    def _():
        ps[...] = psum
        cs[...] = pcnt
        pg[...] = gsum
        cg[...] = gcnt

    @pl.when(i > 0)
    def _():
        ps[...] += psum
        cs[...] += pcnt
        pg[...] += gsum
        cg[...] += gcnt

    @pl.when(i == NPAD // RB - 1)
    def _():
        s = ps[...] / jnp.maximum(cs[...], 1.0)
        g = pg[...] / jnp.maximum(cg[...], 1.0)
        w_s = wsg[...][0:1, :]
        w_g = wsg[...][1:2, :]
        acc = (jnp.sum(s * w_s, axis=1, keepdims=True)
               + jnp.sum(g * w_g, axis=1, keepdims=True))
        out[...] = acc + dep[...] * wdb[0] + wdb[1]


def _row_spec(width):
    return pl.BlockSpec((RB, width), lambda i: (i, 0))


def _full_spec(shape):
    return pl.BlockSpec(shape, lambda i: (0, 0))


_GRID = (NPAD // RB,)

_PREP_OUT_SPECS = [_row_spec(HH), _row_spec(HH), _row_spec(H),
                   _row_spec(HH), _row_spec(HH)] * 2
_PREP_OUT_SHAPE = [jax.ShapeDtypeStruct((NPAD, HH), f32),
                   jax.ShapeDtypeStruct((NPAD, HH), f32),
                   jax.ShapeDtypeStruct((NPAD, H), f32),
                   jax.ShapeDtypeStruct((NPAD, HH), f32),
                   jax.ShapeDtypeStruct((NPAD, HH), f32)] * 2


def _prep0_call(xs, xg, degss, deggg, ws, wg):
    return pl.pallas_call(
        _prep0_body,
        grid=_GRID,
        in_specs=[_row_spec(D), _row_spec(D), _row_spec(16), _row_spec(16),
                  _full_spec((D, 3 * H)), _full_spec((D, 3 * H))],
        out_specs=_PREP_OUT_SPECS,
        out_shape=_PREP_OUT_SHAPE,
    )(xs, xg, degss, deggg, ws, wg)


_COMB_IN_SPECS = ([_row_spec(HH)] * 8
                  + [_row_spec(HH), _row_spec(HH), _row_spec(H)] * 2
                  + [_row_spec(16)] * 4 + [_full_spec((1, H))] * 4)


def _comb_prep_call(*args):
    return pl.pallas_call(
        _comb_prep_body,
        grid=_GRID,
        in_specs=_COMB_IN_SPECS + [_full_spec((H, 3 * H))] * 2,
        out_specs=_PREP_OUT_SPECS,
        out_shape=_PREP_OUT_SHAPE,
    )(*args)


def _comb_pool_call(*args):
    return pl.pallas_call(
        _comb_pool_body,
        grid=_GRID,
        in_specs=_COMB_IN_SPECS + [
            _row_spec(1),
            _row_spec(1),
            _full_spec((B, 1)),
            _full_spec((2, H)),
            pl.BlockSpec(memory_space=pltpu.SMEM),
        ],
        out_specs=pl.BlockSpec((B, 1), lambda i: (0, 0)),
        out_shape=jax.ShapeDtypeStruct((B, 1), f32),
        scratch_shapes=[
            pltpu.VMEM((B, H), f32),
            pltpu.VMEM((B, 1), f32),
            pltpu.VMEM((B, H), f32),
            pltpu.VMEM((B, 1), f32),
        ],
    )(*args)


def _pad_idx(v, fill, rows):
    v = v.astype(i32)
    pad = rows * CH - v.shape[0]
    v = jnp.concatenate([v, jnp.full((pad,), fill, i32)])
    return v.reshape(rows, CH)


def _pad_rows(x):
    return jnp.concatenate(
        [x.astype(f32), jnp.zeros((NPAD - N, x.shape[1]), f32)])


def kernel(x_state, x_goal, ei_ss, ei_gg, ei_sg, ei_gs, batch_state,
           batch_goal, depth,
           W_gcn_s_0, b_gcn_s_0, W_gcn_g_0, b_gcn_g_0, Wl_sg_0, bl_sg_0,
           Wr_sg_0, Wl_gs_0, bl_gs_0, Wr_gs_0,
           W_gcn_s_1, b_gcn_s_1, W_gcn_g_1, b_gcn_g_1, Wl_sg_1, bl_sg_1,
           Wr_sg_1, Wl_gs_1, bl_gs_1, Wr_gs_1, W_out, b_out):
    s_ss = _pad_idx(ei_ss[0], 0, EROWS)
    d_ss = _pad_idx(ei_ss[1], N, EROWS)
    s_gg = _pad_idx(ei_gg[0], 0, EROWS)
    d_gg = _pad_idx(ei_gg[1], N, EROWS)
    s_sg = _pad_idx(ei_sg[0], 0, EROWS)
    d_sg = _pad_idx(ei_sg[1], N, EROWS)
    s_gs = _pad_idx(ei_gs[0], 0, EROWS)
    d_gs = _pad_idx(ei_gs[1], N, EROWS)

    bidx_s = jnp.concatenate(
        [batch_state.astype(i32), jnp.full((NPAD - N,), B, i32)]).reshape(
            NPAD, 1)
    bidx_g = jnp.concatenate(
        [batch_goal.astype(i32), jnp.full((NPAD - N,), B, i32)]).reshape(
            NPAD, 1)

    zeros32 = jnp.zeros((NPAD, HH), f32)
    zeros16 = jnp.zeros((NPAD, 16), f32)

    ws0 = jnp.concatenate([W_gcn_s_0, Wr_gs_0, Wl_sg_0], axis=1)
    wg0 = jnp.concatenate([W_gcn_g_0, Wr_sg_0, Wl_gs_0], axis=1)
    ws1 = jnp.concatenate([W_gcn_s_1, Wr_gs_1, Wl_sg_1], axis=1)
    wg1 = jnp.concatenate([W_gcn_g_1, Wr_sg_1, Wl_gs_1], axis=1)
    bs1_0 = b_gcn_s_0.reshape(1, H)
    bs2_0 = bl_gs_0.reshape(1, H)
    bg1_0 = b_gcn_g_0.reshape(1, H)
    bg2_0 = bl_sg_0.reshape(1, H)
    bs1_1 = b_gcn_s_1.reshape(1, H)
    bs2_1 = bl_gs_1.reshape(1, H)
    bg1_1 = b_gcn_g_1.reshape(1, H)
    bg2_1 = bl_sg_1.reshape(1, H)

    deg_ss, deg_gg, cnt_gs, cnt_sg = _cnt_call(d_ss, d_gg, d_gs, d_sg, zeros16)

    (yl_ss0, yh_ss0, r_s0, yl_sg0, yh_sg0,
     yl_gg0, yh_gg0, r_g0, yl_gs0, yh_gs0) = _prep0_call(
        _pad_rows(x_state), _pad_rows(x_goal), deg_ss, deg_gg, ws0, wg0)

    (al_ss0, ah_ss0, al_gs0, ah_gs0,
     al_gg0, ah_gg0, al_sg0, ah_sg0) = _agg_call(
        yl_ss0, yh_ss0, s_ss, d_ss, yl_gs0, yh_gs0, s_gs, d_gs,
        yl_gg0, yh_gg0, s_gg, d_gg, yl_sg0, yh_sg0, s_sg, d_sg, zeros32)

    (yl_ss1, yh_ss1, r_s1, yl_sg1, yh_sg1,
     yl_gg1, yh_gg1, r_g1, yl_gs1, yh_gs1) = _comb_prep_call(
        al_ss0, ah_ss0, al_gs0, ah_gs0, al_gg0, ah_gg0, al_sg0, ah_sg0,
        yl_ss0, yh_ss0, r_s0, yl_gg0, yh_gg0, r_g0,
        deg_ss, deg_gg, cnt_gs, cnt_sg, bs1_0, bs2_0, bg1_0, bg2_0, ws1, wg1)

    (al_ss1, ah_ss1, al_gs1, ah_gs1,
     al_gg1, ah_gg1, al_sg1, ah_sg1) = _agg_call(
        yl_ss1, yh_ss1, s_ss, d_ss, yl_gs1, yh_gs1, s_gs, d_gs,
        yl_gg1, yh_gg1, s_gg, d_gg, yl_sg1, yh_sg1, s_sg, d_sg, zeros32)

    wsg = jnp.stack([W_out[0:H, 0], W_out[H:2 * H, 0]])
    wdb = jnp.stack([W_out[2 * H, 0], b_out[0]])
    out = _comb_pool_call(
        al_ss1, ah_ss1, al_gs1, ah_gs1, al_gg1, ah_gg1, al_sg1, ah_sg1,
        yl_ss1, yh_ss1, r_s1, yl_gg1, yh_gg1, r_g1,
        deg_ss, deg_gg, cnt_gs, cnt_sg, bs1_1, bs2_1, bg1_1, bg2_1,
        bidx_s, bidx_g, depth.astype(f32).reshape(B, 1), wsg, wdb)
    return out[:, 0]
